# baseline (speedup 1.0000x reference)
import functools

import jax
import jax.numpy as jnp
from jax import lax
from jax.experimental import pallas as pl
from jax.experimental.pallas import tpu as pltpu
from jax.experimental.pallas import tpu_sc as plsc

N, S, C, D1, D2 = 16384, 4096, 3, 64, 128


_BN = 512


def _topk_body(x1_ref, x2_ref, w_ref, idx_ref):
    x1 = x1_ref[...]
    x2 = x2_ref[...]
    mm = lax.dot_general(-2.0 * x1, x2, (((0,), (0,)), ((), ())),
                         precision=lax.Precision.DEFAULT,
                         preferred_element_type=jnp.float32)
    rowsq = jnp.sum(x1 * x1, axis=0).reshape(_BN, 1)
    colsq = jnp.sum(x2 * x2, axis=0).reshape(1, S)
    dist = mm + rowsq
    dist = dist + colsq

    _W = 128
    _F = S // _W
    inf = jnp.float32(jnp.inf)
    a = jnp.full((_BN, _W), inf)
    b = a
    c = a
    zero = jnp.zeros((_BN, _W), jnp.int32)
    ia = zero
    ib = zero
    ic = zero
    for j in range(_F):
        v = lax.slice_in_dim(dist, j * _W, (j + 1) * _W, axis=1)
        iv = jnp.full((_BN, _W), j, jnp.int32)
        le = a <= v
        t = jnp.maximum(a, v)
        a = jnp.minimum(a, v)
        it = jnp.where(le, iv, ia)
        ia = jnp.where(le, ia, iv)
        le = b <= t
        u = jnp.maximum(b, t)
        b = jnp.minimum(b, t)
        iu = jnp.where(le, it, ib)
        ib = jnp.where(le, ib, it)
        le = c <= u
        c = jnp.minimum(c, u)
        ic = jnp.where(le, ic, iu)

    lane = lax.broadcasted_iota(jnp.int32, (_BN, _W), 1)
    V = jnp.concatenate([a, b, c], axis=1)
    COL = jnp.concatenate(
        [ia * _W + lane, ib * _W + lane, ic * _W + lane], axis=1)
    ds = []
    idxs = []
    for _ in range(3):
        m = jnp.min(V, axis=1, keepdims=True)
        cand = jnp.where(V == m, COL, S)
        sel = jnp.min(cand, axis=1, keepdims=True)
        ds.append(m)
        idxs.append(sel)
        V = jnp.where((V == m) & (COL == sel), inf, V)

    r0 = 1.0 / (ds[0] + 1e-8)
    r1 = 1.0 / (ds[1] + 1e-8)
    r2 = 1.0 / (ds[2] + 1e-8)
    norm = r0 + r1 + r2
    w_ref[...] = jnp.concatenate([r0 / norm, r1 / norm, r2 / norm], axis=1)
    idx_ref[...] = jnp.concatenate(
        [i.reshape(1, _BN) for i in idxs], axis=0)


_H = N // 2


def _topk(xyz1, xyz2, half):
    blk0 = half * (_H // _BN)
    return pl.pallas_call(
        _topk_body,
        grid=(_H // _BN,),
        in_specs=[
            pl.BlockSpec((C, _BN), lambda i: (0, i + blk0)),
            pl.BlockSpec((C, S), lambda i: (0, 0)),
        ],
        out_specs=[
            pl.BlockSpec((_BN, 3), lambda i: (i, 0)),
            pl.BlockSpec((3, _BN), lambda i: (0, i)),
        ],
        out_shape=[
            jax.ShapeDtypeStruct((_H, 3), jnp.float32),
            jax.ShapeDtypeStruct((3, _H), jnp.int32),
        ],
    )(xyz1, xyz2)


_NC, _NS = 2, 16
_NW = _NC * _NS
_GCHUNK = 128
_ROWS_PER_W = 3 * _H // _NW
_NCHUNK = _ROWS_PER_W // _GCHUNK


def _gather_body(idx_hbm, tab_hbm, out_hbm,
                 idx0_v, idx1_v, rows0_v, rows1_v, sem0, sem1, osem0, osem1):
    wid = lax.axis_index("s") * _NC + lax.axis_index("c")
    base = wid * _ROWS_PER_W
    idx_v = (idx0_v, idx1_v)
    rows_v = (rows0_v, rows1_v)
    sem = (sem0, sem1)
    osem = (osem0, osem1)
    pltpu.sync_copy(idx_hbm.at[pl.ds(base, _GCHUNK)], idx0_v)
    g = pltpu.async_copy(tab_hbm.at[idx0_v], rows0_v, sem0)
    out_h = [None, None]
    for j in range(_NCHUNK):
        cur = j % 2
        nxt = (j + 1) % 2
        if j + 1 < _NCHUNK:
            off = base + (j + 1) * _GCHUNK
            pltpu.sync_copy(idx_hbm.at[pl.ds(off, _GCHUNK)], idx_v[nxt])
            if out_h[nxt] is not None:
                out_h[nxt].wait()
                out_h[nxt] = None
            g_next = pltpu.async_copy(tab_hbm.at[idx_v[nxt]], rows_v[nxt],
                                      sem[nxt])
        g.wait()
        out_h[cur] = pltpu.async_copy(
            rows_v[cur], out_hbm.at[pl.ds(base + j * _GCHUNK, _GCHUNK)],
            osem[cur])
        if j + 1 < _NCHUNK:
            g = g_next
    for h in out_h:
        if h is not None:
            h.wait()


@functools.cache
def _gather():
    return pl.kernel(
        _gather_body,
        out_type=jax.ShapeDtypeStruct((3 * _H, D2), jnp.float32),
        mesh=plsc.VectorSubcoreMesh(core_axis_name="c", subcore_axis_name="s"),
        scratch_types=[
            pltpu.VMEM((_GCHUNK,), jnp.int32),
            pltpu.VMEM((_GCHUNK,), jnp.int32),
            pltpu.VMEM((_GCHUNK, D2), jnp.float32),
            pltpu.VMEM((_GCHUNK, D2), jnp.float32),
            pltpu.SemaphoreType.DMA,
            pltpu.SemaphoreType.DMA,
            pltpu.SemaphoreType.DMA,
            pltpu.SemaphoreType.DMA,
        ],
    )


_BM = 512


def _mlp_body(g_ref, w_ref, p1_ref, w0a_ref, w0b_ref, w1_ref, p_ref, out_ref):
    g = g_ref[...]
    w = w_ref[...]
    interp = (w[:, 0:1] * g[0] + w[:, 1:2] * g[1] + w[:, 2:3] * g[2])
    h = lax.dot_general(p1_ref[...], w0a_ref[...], (((0,), (0,)), ((), ())),
                        preferred_element_type=jnp.float32)
    h = h + lax.dot_general(interp, w0b_ref[...], (((1,), (0,)), ((), ())),
                            preferred_element_type=jnp.float32)
    p = p_ref[...]
    h = jnp.maximum(h * p[0:1, :] + p[1:2, :], 0.0)
    o = lax.dot_general(w1_ref[...], h, (((0,), (1,)), ((), ())),
                        preferred_element_type=jnp.float32)
    a1 = p[2, :].reshape(128, 1)
    d1 = p[3, :].reshape(128, 1)
    out_ref[...] = jnp.maximum(o * a1 + d1, 0.0)


def _mlp(g, w, points1, w0a, w0b, w1m, params, half):
    blk0 = half * (_H // _BM)
    return pl.pallas_call(
        _mlp_body,
        grid=(_H // _BM,),
        in_specs=[
            pl.BlockSpec((3, _BM, D2), lambda i: (0, i, 0)),
            pl.BlockSpec((_BM, 3), lambda i: (i, 0)),
            pl.BlockSpec((D1, _BM), lambda i: (0, i + blk0)),
            pl.BlockSpec((D1, 128), lambda i: (0, 0)),
            pl.BlockSpec((D2, 128), lambda i: (0, 0)),
            pl.BlockSpec((128, 128), lambda i: (0, 0)),
            pl.BlockSpec((4, 128), lambda i: (0, 0)),
        ],
        out_specs=pl.BlockSpec((128, _BM), lambda i: (0, i)),
        out_shape=jax.ShapeDtypeStruct((128, _H), jnp.float32),
    )(g, w, points1, w0a, w0b, w1m, params)


def kernel(xyz1, xyz2, points1, points2, W0, b0, scale0, bias0, mean0, var0,
           W1, b1, scale1, bias1, mean1, var1):
    table = points2.T

    eps = 1e-5
    a0 = scale0 / jnp.sqrt(var0 + eps)
    d0 = (b0 - mean0) * a0 + bias0
    a1 = scale1 / jnp.sqrt(var1 + eps)
    d1 = (b1 - mean1) * a1 + bias1
    params = jnp.stack([a0, d0, a1, d1])
    w0a, w0b = W0[:D1], W0[D1:]

    outs = []
    gth = _gather()
    w0, idx0 = _topk(xyz1, xyz2, 0)
    g0 = gth(idx0.reshape(3 * _H), table)
    w1, idx1 = _topk(xyz1, xyz2, 1)
    g1 = gth(idx1.reshape(3 * _H), table)
    o0 = _mlp(g0.reshape(3, _H, D2), w0, points1, w0a, w0b, W1, params, 0)
    o1 = _mlp(g1.reshape(3, _H, D2), w1, points1, w0a, w0b, W1, params, 1)
    return jnp.concatenate([o0, o1], axis=1)


# speedup vs baseline: 1.0858x; 1.0858x over previous
import functools

import jax
import jax.numpy as jnp
from jax import lax
from jax.experimental import pallas as pl
from jax.experimental.pallas import tpu as pltpu
from jax.experimental.pallas import tpu_sc as plsc

N, S, C, D1, D2 = 16384, 4096, 3, 64, 128


_BN = 1024


def _topk_body(x1_ref, x2_ref, w_ref, idx_ref):
    x1 = x1_ref[...]
    x2 = x2_ref[...]
    mm = lax.dot_general(-2.0 * x1, x2, (((0,), (0,)), ((), ())),
                         precision=lax.Precision.DEFAULT,
                         preferred_element_type=jnp.float32)
    rowsq = jnp.sum(x1 * x1, axis=0).reshape(_BN, 1)
    colsq = jnp.sum(x2 * x2, axis=0).reshape(1, S)
    dist = mm + rowsq
    dist = dist + colsq

    _W = 128
    _F = S // _W
    inf = jnp.float32(jnp.inf)
    a = jnp.full((_BN, _W), inf)
    b = a
    c = a
    zero = jnp.zeros((_BN, _W), jnp.int32)
    ia = zero
    ib = zero
    ic = zero
    for j in range(_F):
        v = lax.slice_in_dim(dist, j * _W, (j + 1) * _W, axis=1)
        iv = jnp.full((_BN, _W), j, jnp.int32)
        le = a <= v
        t = jnp.maximum(a, v)
        a = jnp.minimum(a, v)
        it = jnp.where(le, iv, ia)
        ia = jnp.where(le, ia, iv)
        le = b <= t
        u = jnp.maximum(b, t)
        b = jnp.minimum(b, t)
        iu = jnp.where(le, it, ib)
        ib = jnp.where(le, ib, it)
        le = c <= u
        c = jnp.minimum(c, u)
        ic = jnp.where(le, ic, iu)

    lane = lax.broadcasted_iota(jnp.int32, (_BN, _W), 1)
    V = jnp.concatenate([a, b, c], axis=1)
    COL = jnp.concatenate(
        [ia * _W + lane, ib * _W + lane, ic * _W + lane], axis=1)
    ds = []
    idxs = []
    for _ in range(3):
        m = jnp.min(V, axis=1, keepdims=True)
        cand = jnp.where(V == m, COL, S)
        sel = jnp.min(cand, axis=1, keepdims=True)
        ds.append(m)
        idxs.append(sel)
        V = jnp.where((V == m) & (COL == sel), inf, V)

    r0 = 1.0 / (ds[0] + 1e-8)
    r1 = 1.0 / (ds[1] + 1e-8)
    r2 = 1.0 / (ds[2] + 1e-8)
    norm = r0 + r1 + r2
    w_ref[...] = jnp.concatenate([r0 / norm, r1 / norm, r2 / norm], axis=1)
    idx_ref[...] = jnp.concatenate(
        [i.reshape(1, _BN) for i in idxs], axis=0)


_H = N // 2


def _topk(xyz1, xyz2, half):
    blk0 = half * (_H // _BN)
    return pl.pallas_call(
        _topk_body,
        grid=(_H // _BN,),
        in_specs=[
            pl.BlockSpec((C, _BN), lambda i: (0, i + blk0)),
            pl.BlockSpec((C, S), lambda i: (0, 0)),
        ],
        out_specs=[
            pl.BlockSpec((_BN, 3), lambda i: (i, 0)),
            pl.BlockSpec((3, _BN), lambda i: (0, i)),
        ],
        out_shape=[
            jax.ShapeDtypeStruct((_H, 3), jnp.float32),
            jax.ShapeDtypeStruct((3, _H), jnp.int32),
        ],
    )(xyz1, xyz2)


_NC, _NS = 2, 16
_NW = _NC * _NS
_GCHUNK = 128
_ROWS_PER_W = 3 * _H // _NW
_NCHUNK = _ROWS_PER_W // _GCHUNK


def _gather_body(idx_hbm, tab_hbm, out_hbm,
                 idx0_v, idx1_v, rows0_v, rows1_v, sem0, sem1, osem0, osem1):
    wid = lax.axis_index("s") * _NC + lax.axis_index("c")
    base = wid * _ROWS_PER_W
    idx_v = (idx0_v, idx1_v)
    rows_v = (rows0_v, rows1_v)
    sem = (sem0, sem1)
    osem = (osem0, osem1)
    pltpu.sync_copy(idx_hbm.at[pl.ds(base, _GCHUNK)], idx0_v)
    g = pltpu.async_copy(tab_hbm.at[idx0_v], rows0_v, sem0)
    out_h = [None, None]
    for j in range(_NCHUNK):
        cur = j % 2
        nxt = (j + 1) % 2
        if j + 1 < _NCHUNK:
            off = base + (j + 1) * _GCHUNK
            pltpu.sync_copy(idx_hbm.at[pl.ds(off, _GCHUNK)], idx_v[nxt])
            if out_h[nxt] is not None:
                out_h[nxt].wait()
                out_h[nxt] = None
            g_next = pltpu.async_copy(tab_hbm.at[idx_v[nxt]], rows_v[nxt],
                                      sem[nxt])
        g.wait()
        out_h[cur] = pltpu.async_copy(
            rows_v[cur], out_hbm.at[pl.ds(base + j * _GCHUNK, _GCHUNK)],
            osem[cur])
        if j + 1 < _NCHUNK:
            g = g_next
    for h in out_h:
        if h is not None:
            h.wait()


@functools.cache
def _gather():
    return pl.kernel(
        _gather_body,
        out_type=jax.ShapeDtypeStruct((3 * _H, D2), jnp.float32),
        mesh=plsc.VectorSubcoreMesh(core_axis_name="c", subcore_axis_name="s"),
        scratch_types=[
            pltpu.VMEM((_GCHUNK,), jnp.int32),
            pltpu.VMEM((_GCHUNK,), jnp.int32),
            pltpu.VMEM((_GCHUNK, D2), jnp.float32),
            pltpu.VMEM((_GCHUNK, D2), jnp.float32),
            pltpu.SemaphoreType.DMA,
            pltpu.SemaphoreType.DMA,
            pltpu.SemaphoreType.DMA,
            pltpu.SemaphoreType.DMA,
        ],
    )


_BM = 1024


def _mlp_body(g_ref, w_ref, p1_ref, w0a_ref, w0b_ref, w1_ref, p_ref, out_ref):
    g = g_ref[...]
    w = w_ref[...]
    interp = (w[:, 0:1] * g[0] + w[:, 1:2] * g[1] + w[:, 2:3] * g[2])
    h = lax.dot_general(p1_ref[...], w0a_ref[...], (((0,), (0,)), ((), ())),
                        preferred_element_type=jnp.float32)
    h = h + lax.dot_general(interp, w0b_ref[...], (((1,), (0,)), ((), ())),
                            preferred_element_type=jnp.float32)
    p = p_ref[...]
    h = jnp.maximum(h * p[0:1, :] + p[1:2, :], 0.0)
    o = lax.dot_general(w1_ref[...], h, (((0,), (1,)), ((), ())),
                        preferred_element_type=jnp.float32)
    a1 = p[2, :].reshape(128, 1)
    d1 = p[3, :].reshape(128, 1)
    out_ref[...] = jnp.maximum(o * a1 + d1, 0.0)


def _mlp(g, w, points1, w0a, w0b, w1m, params, half):
    blk0 = half * (_H // _BM)
    return pl.pallas_call(
        _mlp_body,
        grid=(_H // _BM,),
        in_specs=[
            pl.BlockSpec((3, _BM, D2), lambda i: (0, i, 0)),
            pl.BlockSpec((_BM, 3), lambda i: (i, 0)),
            pl.BlockSpec((D1, _BM), lambda i: (0, i + blk0)),
            pl.BlockSpec((D1, 128), lambda i: (0, 0)),
            pl.BlockSpec((D2, 128), lambda i: (0, 0)),
            pl.BlockSpec((128, 128), lambda i: (0, 0)),
            pl.BlockSpec((4, 128), lambda i: (0, 0)),
        ],
        out_specs=pl.BlockSpec((128, _BM), lambda i: (0, i)),
        out_shape=jax.ShapeDtypeStruct((128, _H), jnp.float32),
    )(g, w, points1, w0a, w0b, w1m, params)


def kernel(xyz1, xyz2, points1, points2, W0, b0, scale0, bias0, mean0, var0,
           W1, b1, scale1, bias1, mean1, var1):
    table = points2.T

    eps = 1e-5
    a0 = scale0 / jnp.sqrt(var0 + eps)
    d0 = (b0 - mean0) * a0 + bias0
    a1 = scale1 / jnp.sqrt(var1 + eps)
    d1 = (b1 - mean1) * a1 + bias1
    params = jnp.stack([a0, d0, a1, d1])
    w0a, w0b = W0[:D1], W0[D1:]

    outs = []
    gth = _gather()
    w0, idx0 = _topk(xyz1, xyz2, 0)
    g0 = gth(idx0.reshape(3 * _H), table)
    w1, idx1 = _topk(xyz1, xyz2, 1)
    g1 = gth(idx1.reshape(3 * _H), table)
    o0 = _mlp(g0.reshape(3, _H, D2), w0, points1, w0a, w0b, W1, params, 0)
    o1 = _mlp(g1.reshape(3, _H, D2), w1, points1, w0a, w0b, W1, params, 1)
    return jnp.concatenate([o0, o1], axis=1)


# speedup vs baseline: 1.1030x; 1.0158x over previous
import functools

import jax
import jax.numpy as jnp
from jax import lax
from jax.experimental import pallas as pl
from jax.experimental.pallas import tpu as pltpu
from jax.experimental.pallas import tpu_sc as plsc

N, S, C, D1, D2 = 16384, 4096, 3, 64, 128


_BN = 1024


def _topk_body(x1_ref, x2_ref, w_ref, idx_ref):
    x1 = x1_ref[...]
    x2 = x2_ref[...]
    mm = lax.dot_general(-2.0 * x1, x2, (((0,), (0,)), ((), ())),
                         precision=lax.Precision.DEFAULT,
                         preferred_element_type=jnp.float32)
    rowsq = jnp.sum(x1 * x1, axis=0).reshape(_BN, 1)
    colsq = jnp.sum(x2 * x2, axis=0).reshape(1, S)
    dist = mm + rowsq
    dist = dist + colsq

    _W = 128
    _F = S // _W
    inf = jnp.float32(jnp.inf)
    a = jnp.full((_BN, _W), inf)
    b = a
    c = a
    zero = jnp.zeros((_BN, _W), jnp.int32)
    ia = zero
    ib = zero
    ic = zero
    for j in range(_F):
        v = lax.slice_in_dim(dist, j * _W, (j + 1) * _W, axis=1)
        iv = jnp.full((_BN, _W), j, jnp.int32)
        le = a <= v
        t = jnp.maximum(a, v)
        a = jnp.minimum(a, v)
        it = jnp.where(le, iv, ia)
        ia = jnp.where(le, ia, iv)
        le = b <= t
        u = jnp.maximum(b, t)
        b = jnp.minimum(b, t)
        iu = jnp.where(le, it, ib)
        ib = jnp.where(le, ib, it)
        le = c <= u
        c = jnp.minimum(c, u)
        ic = jnp.where(le, ic, iu)

    lane = lax.broadcasted_iota(jnp.int32, (_BN, _W), 1)
    ca = ia * _W + lane
    cb = ib * _W + lane
    cc = ic * _W + lane
    ds = []
    idxs = []
    for k in range(3):
        m = jnp.min(a, axis=1, keepdims=True)
        eqm = a == m
        col = jnp.min(jnp.where(eqm, ca, S), axis=1, keepdims=True)
        ds.append(m)
        idxs.append(col)
        if k < 2:
            pop = eqm & (ca == col)
            a = jnp.where(pop, b, a)
            ca = jnp.where(pop, cb, ca)
            b = jnp.where(pop, c, b)
            cb = jnp.where(pop, cc, cb)
            c = jnp.where(pop, inf, c)

    r0 = 1.0 / (ds[0] + 1e-8)
    r1 = 1.0 / (ds[1] + 1e-8)
    r2 = 1.0 / (ds[2] + 1e-8)
    norm = r0 + r1 + r2
    w_ref[...] = jnp.concatenate([r0 / norm, r1 / norm, r2 / norm], axis=1)
    idx_ref[...] = jnp.concatenate(
        [i.reshape(1, _BN) for i in idxs], axis=0)


_H = N // 2


def _topk(xyz1, xyz2, half):
    blk0 = half * (_H // _BN)
    return pl.pallas_call(
        _topk_body,
        grid=(_H // _BN,),
        in_specs=[
            pl.BlockSpec((C, _BN), lambda i: (0, i + blk0)),
            pl.BlockSpec((C, S), lambda i: (0, 0)),
        ],
        out_specs=[
            pl.BlockSpec((_BN, 3), lambda i: (i, 0)),
            pl.BlockSpec((3, _BN), lambda i: (0, i)),
        ],
        out_shape=[
            jax.ShapeDtypeStruct((_H, 3), jnp.float32),
            jax.ShapeDtypeStruct((3, _H), jnp.int32),
        ],
    )(xyz1, xyz2)


_NC, _NS = 2, 16
_NW = _NC * _NS
_GCHUNK = 128
_ROWS_PER_W = 3 * _H // _NW
_NCHUNK = _ROWS_PER_W // _GCHUNK


def _gather_body(idx_hbm, tab_hbm, out_hbm,
                 idx0_v, idx1_v, rows0_v, rows1_v, sem0, sem1, osem0, osem1):
    wid = lax.axis_index("s") * _NC + lax.axis_index("c")
    base = wid * _ROWS_PER_W
    idx_v = (idx0_v, idx1_v)
    rows_v = (rows0_v, rows1_v)
    sem = (sem0, sem1)
    osem = (osem0, osem1)
    pltpu.sync_copy(idx_hbm.at[pl.ds(base, _GCHUNK)], idx0_v)
    g = pltpu.async_copy(tab_hbm.at[idx0_v], rows0_v, sem0)
    out_h = [None, None]
    for j in range(_NCHUNK):
        cur = j % 2
        nxt = (j + 1) % 2
        if j + 1 < _NCHUNK:
            off = base + (j + 1) * _GCHUNK
            pltpu.sync_copy(idx_hbm.at[pl.ds(off, _GCHUNK)], idx_v[nxt])
            if out_h[nxt] is not None:
                out_h[nxt].wait()
                out_h[nxt] = None
            g_next = pltpu.async_copy(tab_hbm.at[idx_v[nxt]], rows_v[nxt],
                                      sem[nxt])
        g.wait()
        out_h[cur] = pltpu.async_copy(
            rows_v[cur], out_hbm.at[pl.ds(base + j * _GCHUNK, _GCHUNK)],
            osem[cur])
        if j + 1 < _NCHUNK:
            g = g_next
    for h in out_h:
        if h is not None:
            h.wait()


@functools.cache
def _gather():
    return pl.kernel(
        _gather_body,
        out_type=jax.ShapeDtypeStruct((3 * _H, D2), jnp.float32),
        mesh=plsc.VectorSubcoreMesh(core_axis_name="c", subcore_axis_name="s"),
        scratch_types=[
            pltpu.VMEM((_GCHUNK,), jnp.int32),
            pltpu.VMEM((_GCHUNK,), jnp.int32),
            pltpu.VMEM((_GCHUNK, D2), jnp.float32),
            pltpu.VMEM((_GCHUNK, D2), jnp.float32),
            pltpu.SemaphoreType.DMA,
            pltpu.SemaphoreType.DMA,
            pltpu.SemaphoreType.DMA,
            pltpu.SemaphoreType.DMA,
        ],
    )


_BM = 1024


def _mlp_body(g_ref, w_ref, p1_ref, w0a_ref, w0b_ref, w1_ref, p_ref, out_ref):
    g = g_ref[...]
    w = w_ref[...]
    interp = (w[:, 0:1] * g[0] + w[:, 1:2] * g[1] + w[:, 2:3] * g[2])
    h = lax.dot_general(p1_ref[...], w0a_ref[...], (((0,), (0,)), ((), ())),
                        preferred_element_type=jnp.float32)
    h = h + lax.dot_general(interp, w0b_ref[...], (((1,), (0,)), ((), ())),
                            preferred_element_type=jnp.float32)
    p = p_ref[...]
    h = jnp.maximum(h * p[0:1, :] + p[1:2, :], 0.0)
    o = lax.dot_general(w1_ref[...], h, (((0,), (1,)), ((), ())),
                        preferred_element_type=jnp.float32)
    a1 = p[2, :].reshape(128, 1)
    d1 = p[3, :].reshape(128, 1)
    out_ref[...] = jnp.maximum(o * a1 + d1, 0.0)


def _mlp(g, w, points1, w0a, w0b, w1m, params, half):
    blk0 = half * (_H // _BM)
    return pl.pallas_call(
        _mlp_body,
        grid=(_H // _BM,),
        in_specs=[
            pl.BlockSpec((3, _BM, D2), lambda i: (0, i, 0)),
            pl.BlockSpec((_BM, 3), lambda i: (i, 0)),
            pl.BlockSpec((D1, _BM), lambda i: (0, i + blk0)),
            pl.BlockSpec((D1, 128), lambda i: (0, 0)),
            pl.BlockSpec((D2, 128), lambda i: (0, 0)),
            pl.BlockSpec((128, 128), lambda i: (0, 0)),
            pl.BlockSpec((4, 128), lambda i: (0, 0)),
        ],
        out_specs=pl.BlockSpec((128, _BM), lambda i: (0, i)),
        out_shape=jax.ShapeDtypeStruct((128, _H), jnp.float32),
    )(g, w, points1, w0a, w0b, w1m, params)


def kernel(xyz1, xyz2, points1, points2, W0, b0, scale0, bias0, mean0, var0,
           W1, b1, scale1, bias1, mean1, var1):
    table = points2.T

    eps = 1e-5
    a0 = scale0 / jnp.sqrt(var0 + eps)
    d0 = (b0 - mean0) * a0 + bias0
    a1 = scale1 / jnp.sqrt(var1 + eps)
    d1 = (b1 - mean1) * a1 + bias1
    params = jnp.stack([a0, d0, a1, d1])
    w0a, w0b = W0[:D1], W0[D1:]

    outs = []
    gth = _gather()
    w0, idx0 = _topk(xyz1, xyz2, 0)
    g0 = gth(idx0.reshape(3 * _H), table)
    w1, idx1 = _topk(xyz1, xyz2, 1)
    g1 = gth(idx1.reshape(3 * _H), table)
    o0 = _mlp(g0.reshape(3, _H, D2), w0, points1, w0a, w0b, W1, params, 0)
    o1 = _mlp(g1.reshape(3, _H, D2), w1, points1, w0a, w0b, W1, params, 1)
    return jnp.concatenate([o0, o1], axis=1)
